# Initial kernel scaffold; baseline (speedup 1.0000x reference)
import jax
import jax.numpy as jnp
from jax.experimental import pallas as pl


def kernel(args, limits, W1, b1, W2, b2, ln_g, ln_b):
    raise NotImplementedError("write your pallas kernel here")


# speedup vs baseline: 2.2894x; 2.2894x over previous
import functools

import jax
import jax.numpy as jnp
from jax.experimental import pallas as pl
from jax.experimental.pallas import tpu as pltpu


def _round_up(x, m):
    return (x + m - 1) // m * m


def _bits(maxval):
    return [1 << j for j in reversed(range(max(1, int(maxval).bit_length())))]


def _mlp_body(x_ref, w1_ref, b1_ref, w2_ref, b2_ref, g_ref, bb_ref, o_ref):
    x = x_ref[...]
    h = jax.lax.dot_general(x, w1_ref[...], (((1,), (1,)), ((), ())),
                            preferred_element_type=jnp.float32)
    h = jnp.maximum(h + b1_ref[...], 0.0)
    y = jax.lax.dot_general(h, w2_ref[...], (((1,), (1,)), ((), ())),
                            preferred_element_type=jnp.float32)
    y = y + b2_ref[...]
    mu = jnp.mean(y, axis=-1, keepdims=True)
    var = jnp.mean((y - mu) ** 2, axis=-1, keepdims=True)
    o_ref[...] = (y - mu) * jax.lax.rsqrt(var + 1e-5) * g_ref[...] + bb_ref[...]


def _mlp_level(xmat, w1, b1, w2, b2, g, bb, tp):
    p, d2 = xmat.shape
    dim = d2 // 2
    grid = (p + tp - 1) // tp
    return pl.pallas_call(
        _mlp_body,
        grid=(grid,),
        in_specs=[
            pl.BlockSpec((tp, d2), lambda i: (i, 0)),
            pl.BlockSpec((d2, d2), lambda i: (0, 0)),
            pl.BlockSpec((1, d2), lambda i: (0, 0)),
            pl.BlockSpec((dim, d2), lambda i: (0, 0)),
            pl.BlockSpec((1, dim), lambda i: (0, 0)),
            pl.BlockSpec((1, dim), lambda i: (0, 0)),
            pl.BlockSpec((1, dim), lambda i: (0, 0)),
        ],
        out_specs=pl.BlockSpec((tp, dim), lambda i: (i, 0)),
        out_shape=jax.ShapeDtypeStruct((p, dim), jnp.float32),
    )(xmat, w1, b1.reshape(1, d2), w2, b2.reshape(1, dim),
      g.reshape(1, dim), bb.reshape(1, dim))


def _start_or_wait(plans, out_ref, sems, start):
    for idx, (cond, src, s, d, b) in enumerate(plans):
        @pl.when(cond)
        def _(src=src, s=s, d=d, b=b, idx=idx):
            cp = pltpu.make_async_copy(
                src.at[pl.ds(s, b)], out_ref.at[pl.ds(d, b)], sems.at[idx])
            if start:
                cp.start()
            else:
                cp.wait()


def _pack_body(soff_ref, len_ref, doff_ref, src_ref, out_ref, sems, *,
               nseg, bits):
    plans = []
    for i in range(nseg):
        nrows = len_ref[i]
        cnt = nrows + (nrows & 1)
        s = soff_ref[i]
        d = doff_ref[i]
        for b in bits:
            take = cnt & b
            plans.append(((take != 0), src_ref, s, d, b))
            s = s + take
            d = d + take
    _start_or_wait(plans, out_ref, sems, True)
    _start_or_wait(plans, out_ref, sems, False)


def _assemble_body(off_ref, len_ref, noff_ref, y_ref, buf_ref, out_ref, sems,
                   *, nseg, bits):
    plans = []
    for i in range(nseg):
        ln = len_ref[i]
        o = off_ref[i]
        no = noff_ref[i]
        pairs = ln // 2
        odd = ln & 1
        plans.append(((odd == 1), buf_ref, jnp.maximum(o + ln - 1, 0), no, 1))
        ys = o // 2
        d = no + odd
        for b in bits:
            take = pairs & b
            plans.append(((take != 0), y_ref, ys, d, b))
            ys = ys + take
            d = d + take
    _start_or_wait(plans, out_ref, sems, True)
    _start_or_wait(plans, out_ref, sems, False)


def _extract_body(off_ref, len_ref, buf_ref, out_ref, scratch, sems, *, nseg):
    for i in range(nseg):
        pltpu.make_async_copy(buf_ref.at[pl.ds(off_ref[i], 1)],
                              scratch.at[pl.ds(i, 1)], sems.at[i]).start()
    for i in range(nseg):
        pltpu.make_async_copy(buf_ref.at[pl.ds(off_ref[i], 1)],
                              scratch.at[pl.ds(i, 1)], sems.at[i]).wait()
    for i in range(nseg):
        nz = len_ref[i] > 0

        @pl.when(nz)
        def _(i=i):
            out_ref[i] = scratch[i]

        @pl.when(jnp.logical_not(nz))
        def _(i=i):
            out_ref[i] = jnp.zeros_like(scratch[i])


def kernel(args, limits, W1, b1, W2, b2, ln_g, ln_b):
    n, dim = args.shape
    d2 = 2 * dim
    nseg = limits.shape[0] - 1
    levels = max(1, (n - 1).bit_length())
    maxlen = n - 1

    limits = limits.astype(jnp.int32)
    lens0 = limits[1:] - limits[:-1]

    lvl_lens, lvl_offs, lvl_sizes = [], [], []
    ln = lens0
    for k in range(levels + 1):
        padded = ln + (ln & 1)
        offs = jnp.concatenate(
            [jnp.zeros((1,), jnp.int32), jnp.cumsum(padded)[:-1].astype(jnp.int32)])
        lvl_lens.append(ln)
        lvl_offs.append(offs)
        lvl_sizes.append(_round_up(-(-maxlen // (1 << k)) + 2 * nseg, 16))
        ln = (ln + 1) // 2

    anyspec = pl.BlockSpec(memory_space=pl.ANY)
    smem = pl.BlockSpec(memory_space=pltpu.SMEM)

    lanes = 128 if dim % 128 == 0 else 1
    sub = dim // lanes

    bits0 = _bits(lvl_sizes[0])
    buf = pl.pallas_call(
        functools.partial(_pack_body, nseg=nseg, bits=bits0),
        in_specs=[smem, smem, smem, anyspec],
        out_specs=anyspec,
        out_shape=jax.ShapeDtypeStruct((lvl_sizes[0], sub, lanes), jnp.float32),
        scratch_shapes=[pltpu.SemaphoreType.DMA((nseg * len(bits0),))],
    )(limits[:-1], lens0, lvl_offs[0], args.reshape(n, sub, lanes))

    for k in range(levels):
        p = lvl_sizes[k] // 2
        tp = min(256, p)
        y = _mlp_level(buf.reshape(p, d2), W1, b1, W2, b2, ln_g, ln_b, tp)
        bits = _bits(p)
        nplans = nseg * (len(bits) + 1)
        buf = pl.pallas_call(
            functools.partial(_assemble_body, nseg=nseg, bits=bits),
            in_specs=[smem, smem, smem, anyspec, anyspec],
            out_specs=anyspec,
            out_shape=jax.ShapeDtypeStruct((lvl_sizes[k + 1], sub, lanes),
                                           jnp.float32),
            scratch_shapes=[pltpu.SemaphoreType.DMA((nplans,))],
        )(lvl_offs[k], lvl_lens[k], lvl_offs[k + 1],
          y.reshape(p, sub, lanes), buf)

    out = pl.pallas_call(
        functools.partial(_extract_body, nseg=nseg),
        in_specs=[smem, smem, anyspec],
        out_shape=jax.ShapeDtypeStruct((nseg, sub, lanes), jnp.float32),
        scratch_shapes=[pltpu.VMEM((nseg, sub, lanes), jnp.float32),
                        pltpu.SemaphoreType.DMA((nseg,))],
    )(lvl_offs[levels], lens0, buf)
    return out.reshape(nseg, dim)


# speedup vs baseline: 3.7051x; 1.6184x over previous
import functools

import jax
import jax.numpy as jnp
from jax.experimental import pallas as pl
from jax.experimental.pallas import tpu as pltpu


def _round_up(x, m):
    return (x + m - 1) // m * m


def _bits(maxval):
    return [1 << j for j in reversed(range(max(1, int(maxval).bit_length())))]


def _start_or_wait(plans, dst_ref, sems, start):
    for idx, (cond, src, s, d, b) in enumerate(plans):
        @pl.when(cond)
        def _(src=src, s=s, d=d, b=b, idx=idx):
            cp = pltpu.make_async_copy(
                src.at[pl.ds(s, b)], dst_ref.at[pl.ds(d, b)], sems.at[idx])
            if start:
                cp.start()
            else:
                cp.wait()


def _pack_body(soff_ref, len_ref, doff_ref, src_ref, out_ref, sems, *,
               nseg, bits):
    plans = []
    for i in range(nseg):
        nrows = len_ref[i]
        cnt = nrows + (nrows & 1)
        s = soff_ref[i]
        d = doff_ref[i]
        for b in bits:
            take = cnt & b
            plans.append(((take != 0), src_ref, s, d, b))
            s = s + take
            d = d + take
    _start_or_wait(plans, out_ref, sems, True)
    _start_or_wait(plans, out_ref, sems, False)


def _level_body(off_ref, len_ref, noff_ref, x_ref, w1_ref, b1_ref, w2_ref,
                b2_ref, g_ref, bb_ref, buf_ref, nbuf_ref, y3, sems, lsems, *,
                nseg, tp, bits, sub, lanes):
    x = x_ref[...]
    h = jax.lax.dot_general(x, w1_ref[...], (((1,), (1,)), ((), ())),
                            preferred_element_type=jnp.float32)
    h = jnp.maximum(h + b1_ref[...], 0.0)
    y = jax.lax.dot_general(h, w2_ref[...], (((1,), (1,)), ((), ())),
                            preferred_element_type=jnp.float32)
    y = y + b2_ref[...]
    mu = jnp.mean(y, axis=-1, keepdims=True)
    var = jnp.mean((y - mu) ** 2, axis=-1, keepdims=True)
    y = (y - mu) * jax.lax.rsqrt(var + 1e-5) * g_ref[...] + bb_ref[...]

    for s in range(sub):
        y3[:, s, :] = y[:, s * lanes:(s + 1) * lanes]

    i = pl.program_id(0)
    t0 = i * tp
    plans = []
    for s in range(nseg):
        po = off_ref[s] // 2
        pairs = len_ref[s] // 2
        odd = len_ref[s] & 1
        a = jnp.maximum(t0, po)
        b = jnp.minimum(t0 + tp, po + pairs)
        cnt = jnp.maximum(b - a, 0)
        src = a - t0
        d = noff_ref[s] + odd + (a - po)
        for bit in bits:
            take = cnt & bit
            plans.append(((take != 0), y3, src, d, bit))
            src = src + take
            d = d + take
    _start_or_wait(plans, nbuf_ref, sems, True)

    lplans = []
    for s in range(nseg):
        ln = len_ref[s]
        lplans.append(((ln & 1) == 1, buf_ref,
                       jnp.maximum(off_ref[s] + ln - 1, 0), noff_ref[s], 1))

    @pl.when(i == 0)
    def _():
        _start_or_wait(lplans, nbuf_ref, lsems, True)
        _start_or_wait(lplans, nbuf_ref, lsems, False)

    _start_or_wait(plans, nbuf_ref, sems, False)


def _run_level(xmat, buf3, w1, b1, w2, b2, g, bb, offs, lens, noffs,
               out_rows, tp, sub, lanes):
    p, d2 = xmat.shape
    dim = d2 // 2
    grid = (p + tp - 1) // tp
    bits = _bits(tp)
    nplans = 8 * len(bits)
    nseg = lens.shape[0]
    return pl.pallas_call(
        functools.partial(_level_body, nseg=nseg, tp=tp, bits=bits,
                          sub=sub, lanes=lanes),
        grid=(grid,),
        in_specs=[
            pl.BlockSpec(memory_space=pltpu.SMEM),
            pl.BlockSpec(memory_space=pltpu.SMEM),
            pl.BlockSpec(memory_space=pltpu.SMEM),
            pl.BlockSpec((tp, d2), lambda i: (i, 0)),
            pl.BlockSpec((d2, d2), lambda i: (0, 0)),
            pl.BlockSpec((1, d2), lambda i: (0, 0)),
            pl.BlockSpec((dim, d2), lambda i: (0, 0)),
            pl.BlockSpec((1, dim), lambda i: (0, 0)),
            pl.BlockSpec((1, dim), lambda i: (0, 0)),
            pl.BlockSpec((1, dim), lambda i: (0, 0)),
            pl.BlockSpec(memory_space=pl.ANY),
        ],
        out_specs=pl.BlockSpec(memory_space=pl.ANY),
        out_shape=jax.ShapeDtypeStruct((out_rows, sub, lanes), jnp.float32),
        scratch_shapes=[pltpu.VMEM((tp, sub, lanes), jnp.float32),
                        pltpu.SemaphoreType.DMA((nseg * len(bits),)),
                        pltpu.SemaphoreType.DMA((nseg,))],
    )(offs, lens, noffs, xmat, w1, b1.reshape(1, d2), w2,
      b2.reshape(1, dim), g.reshape(1, dim), bb.reshape(1, dim), buf3)


def _extract_body(off_ref, len_ref, buf_ref, out_ref, scratch, sems, *, nseg):
    for i in range(nseg):
        pltpu.make_async_copy(buf_ref.at[pl.ds(off_ref[i], 1)],
                              scratch.at[pl.ds(i, 1)], sems.at[i]).start()
    for i in range(nseg):
        pltpu.make_async_copy(buf_ref.at[pl.ds(off_ref[i], 1)],
                              scratch.at[pl.ds(i, 1)], sems.at[i]).wait()
    for i in range(nseg):
        nz = len_ref[i] > 0

        @pl.when(nz)
        def _(i=i):
            out_ref[i] = scratch[i]

        @pl.when(jnp.logical_not(nz))
        def _(i=i):
            out_ref[i] = jnp.zeros_like(scratch[i])


def kernel(args, limits, W1, b1, W2, b2, ln_g, ln_b):
    n, dim = args.shape
    d2 = 2 * dim
    nseg = limits.shape[0] - 1
    levels = max(1, (n - 1).bit_length())
    maxlen = n - 1

    limits = limits.astype(jnp.int32)
    lens0 = limits[1:] - limits[:-1]

    lvl_lens, lvl_offs, lvl_sizes = [], [], []
    ln = lens0
    for k in range(levels + 1):
        padded = ln + (ln & 1)
        offs = jnp.concatenate(
            [jnp.zeros((1,), jnp.int32), jnp.cumsum(padded)[:-1].astype(jnp.int32)])
        lvl_lens.append(ln)
        lvl_offs.append(offs)
        lvl_sizes.append(_round_up(-(-maxlen // (1 << k)) + 2 * nseg, 16))
        ln = (ln + 1) // 2

    anyspec = pl.BlockSpec(memory_space=pl.ANY)
    smem = pl.BlockSpec(memory_space=pltpu.SMEM)

    lanes = 128 if dim % 128 == 0 else 1
    sub = dim // lanes

    bits0 = _bits(lvl_sizes[0])
    buf = pl.pallas_call(
        functools.partial(_pack_body, nseg=nseg, bits=bits0),
        in_specs=[smem, smem, smem, anyspec],
        out_specs=anyspec,
        out_shape=jax.ShapeDtypeStruct((lvl_sizes[0], sub, lanes), jnp.float32),
        scratch_shapes=[pltpu.SemaphoreType.DMA((nseg * len(bits0),))],
    )(limits[:-1], lens0, lvl_offs[0], args.reshape(n, sub, lanes))

    for k in range(levels):
        p = lvl_sizes[k] // 2
        tp = min(256, p)
        buf = _run_level(buf.reshape(p, d2), buf, W1, b1, W2, b2, ln_g, ln_b,
                         lvl_offs[k], lvl_lens[k], lvl_offs[k + 1],
                         lvl_sizes[k + 1], tp, sub, lanes)

    out = pl.pallas_call(
        functools.partial(_extract_body, nseg=nseg),
        in_specs=[smem, smem, anyspec],
        out_shape=jax.ShapeDtypeStruct((nseg, sub, lanes), jnp.float32),
        scratch_shapes=[pltpu.VMEM((nseg, sub, lanes), jnp.float32),
                        pltpu.SemaphoreType.DMA((nseg,))],
    )(lvl_offs[levels], lens0, buf)
    return out.reshape(nseg, dim)


# speedup vs baseline: 3.9923x; 1.0775x over previous
import functools

import jax
import jax.numpy as jnp
from jax.experimental import pallas as pl
from jax.experimental.pallas import tpu as pltpu


def _round_up(x, m):
    return (x + m - 1) // m * m


def _bits(maxval):
    return [1 << j for j in reversed(range(max(1, int(maxval).bit_length())))]


def _start_or_wait(plans, dst_ref, sems, start):
    for idx, (cond, src, s, d, b) in enumerate(plans):
        @pl.when(cond)
        def _(src=src, s=s, d=d, b=b, idx=idx):
            cp = pltpu.make_async_copy(
                src.at[pl.ds(s, b)], dst_ref.at[pl.ds(d, b)], sems.at[idx])
            if start:
                cp.start()
            else:
                cp.wait()


def _pack_body(soff_ref, len_ref, doff_ref, src_ref, out_ref, sems, *,
               nseg, bits):
    plans = []
    for i in range(nseg):
        nrows = len_ref[i]
        cnt = nrows + (nrows & 1)
        s = soff_ref[i]
        d = doff_ref[i]
        for b in bits:
            take = cnt & b
            plans.append(((take != 0), src_ref, s, d, b))
            s = s + take
            d = d + take
    _start_or_wait(plans, out_ref, sems, True)
    _start_or_wait(plans, out_ref, sems, False)


def _level_body(off_ref, len_ref, noff_ref, x_ref, w1_ref, b1_ref, w2_ref,
                b2_ref, g_ref, bb_ref, buf_ref, nbuf_ref, y3, sems, lsems, *,
                nseg, tp, bits, sub, lanes):
    x = x_ref[...].reshape(tp, 2 * sub * lanes)
    h = jax.lax.dot_general(x, w1_ref[...], (((1,), (1,)), ((), ())),
                            preferred_element_type=jnp.float32)
    h = jnp.maximum(h + b1_ref[...], 0.0)
    y = jax.lax.dot_general(h, w2_ref[...], (((1,), (1,)), ((), ())),
                            preferred_element_type=jnp.float32)
    y = y + b2_ref[...]
    mu = jnp.mean(y, axis=-1, keepdims=True)
    var = jnp.mean((y - mu) ** 2, axis=-1, keepdims=True)
    y = (y - mu) * jax.lax.rsqrt(var + 1e-5) * g_ref[...] + bb_ref[...]

    for s in range(sub):
        y3[:, s, :] = y[:, s * lanes:(s + 1) * lanes]

    i = pl.program_id(0)
    t0 = i * tp
    plans = []
    for s in range(nseg):
        po = off_ref[s] // 2
        pairs = len_ref[s] // 2
        odd = len_ref[s] & 1
        a = jnp.maximum(t0, po)
        b = jnp.minimum(t0 + tp, po + pairs)
        cnt = jnp.maximum(b - a, 0)
        src = a - t0
        d = noff_ref[s] + odd + (a - po)
        for bit in bits:
            take = cnt & bit
            plans.append(((take != 0), y3, src, d, bit))
            src = src + take
            d = d + take
    _start_or_wait(plans, nbuf_ref, sems, True)

    lplans = []
    for s in range(nseg):
        ln = len_ref[s]
        lplans.append(((ln & 1) == 1, buf_ref,
                       jnp.maximum(off_ref[s] + ln - 1, 0), noff_ref[s], 1))

    @pl.when(i == 0)
    def _():
        _start_or_wait(lplans, nbuf_ref, lsems, True)
        _start_or_wait(lplans, nbuf_ref, lsems, False)

    _start_or_wait(plans, nbuf_ref, sems, False)


def _run_level(buf3, w1, b1, w2, b2, g, bb, offs, lens, noffs,
               out_rows, tp, sub, lanes):
    rows = buf3.shape[0]
    p = rows // 2
    dim = sub * lanes
    d2 = 2 * dim
    grid = (p + tp - 1) // tp
    bits = _bits(tp)
    nseg = lens.shape[0]
    return pl.pallas_call(
        functools.partial(_level_body, nseg=nseg, tp=tp, bits=bits,
                          sub=sub, lanes=lanes),
        grid=(grid,),
        in_specs=[
            pl.BlockSpec(memory_space=pltpu.SMEM),
            pl.BlockSpec(memory_space=pltpu.SMEM),
            pl.BlockSpec(memory_space=pltpu.SMEM),
            pl.BlockSpec((2 * tp, sub, lanes), lambda i: (i, 0, 0)),
            pl.BlockSpec((d2, d2), lambda i: (0, 0)),
            pl.BlockSpec((1, d2), lambda i: (0, 0)),
            pl.BlockSpec((dim, d2), lambda i: (0, 0)),
            pl.BlockSpec((1, dim), lambda i: (0, 0)),
            pl.BlockSpec((1, dim), lambda i: (0, 0)),
            pl.BlockSpec((1, dim), lambda i: (0, 0)),
            pl.BlockSpec(memory_space=pl.ANY),
        ],
        out_specs=pl.BlockSpec(memory_space=pl.ANY),
        out_shape=jax.ShapeDtypeStruct((out_rows, sub, lanes), jnp.float32),
        scratch_shapes=[pltpu.VMEM((tp, sub, lanes), jnp.float32),
                        pltpu.SemaphoreType.DMA((nseg * len(bits),)),
                        pltpu.SemaphoreType.DMA((nseg,))],
    )(offs, lens, noffs, buf3, w1, b1.reshape(1, d2), w2,
      b2.reshape(1, dim), g.reshape(1, dim), bb.reshape(1, dim), buf3)


def _extract_body(off_ref, len_ref, buf_ref, out_ref, scratch, sems, *, nseg):
    for i in range(nseg):
        pltpu.make_async_copy(buf_ref.at[pl.ds(off_ref[i], 1)],
                              scratch.at[pl.ds(i, 1)], sems.at[i]).start()
    for i in range(nseg):
        pltpu.make_async_copy(buf_ref.at[pl.ds(off_ref[i], 1)],
                              scratch.at[pl.ds(i, 1)], sems.at[i]).wait()
    for i in range(nseg):
        nz = len_ref[i] > 0

        @pl.when(nz)
        def _(i=i):
            out_ref[i] = scratch[i]

        @pl.when(jnp.logical_not(nz))
        def _(i=i):
            out_ref[i] = jnp.zeros_like(scratch[i])


def kernel(args, limits, W1, b1, W2, b2, ln_g, ln_b):
    n, dim = args.shape
    d2 = 2 * dim
    nseg = limits.shape[0] - 1
    levels = max(1, (n - 1).bit_length())
    maxlen = n - 1

    limits = limits.astype(jnp.int32)
    lens0 = limits[1:] - limits[:-1]

    lvl_lens, lvl_offs, lvl_sizes = [], [], []
    ln = lens0
    for k in range(levels + 1):
        padded = ln + (ln & 1)
        offs = jnp.concatenate(
            [jnp.zeros((1,), jnp.int32), jnp.cumsum(padded)[:-1].astype(jnp.int32)])
        lvl_lens.append(ln)
        lvl_offs.append(offs)
        lvl_sizes.append(_round_up(-(-maxlen // (1 << k)) + 2 * nseg, 16))
        ln = (ln + 1) // 2

    anyspec = pl.BlockSpec(memory_space=pl.ANY)
    smem = pl.BlockSpec(memory_space=pltpu.SMEM)

    lanes = 128 if dim % 128 == 0 else 1
    sub = dim // lanes

    bits0 = _bits(lvl_sizes[0])
    buf = pl.pallas_call(
        functools.partial(_pack_body, nseg=nseg, bits=bits0),
        in_specs=[smem, smem, smem, anyspec],
        out_specs=anyspec,
        out_shape=jax.ShapeDtypeStruct((lvl_sizes[0], sub, lanes), jnp.float32),
        scratch_shapes=[pltpu.SemaphoreType.DMA((nseg * len(bits0),))],
    )(limits[:-1], lens0, lvl_offs[0], args.reshape(n, sub, lanes))

    for k in range(levels):
        p = lvl_sizes[k] // 2
        tp = min(256, p)
        buf = _run_level(buf, W1, b1, W2, b2, ln_g, ln_b,
                         lvl_offs[k], lvl_lens[k], lvl_offs[k + 1],
                         lvl_sizes[k + 1], tp, sub, lanes)

    out = pl.pallas_call(
        functools.partial(_extract_body, nseg=nseg),
        in_specs=[smem, smem, anyspec],
        out_shape=jax.ShapeDtypeStruct((nseg, sub, lanes), jnp.float32),
        scratch_shapes=[pltpu.VMEM((nseg, sub, lanes), jnp.float32),
                        pltpu.SemaphoreType.DMA((nseg,))],
    )(lvl_offs[levels], lens0, buf)
    return out.reshape(nseg, dim)


# speedup vs baseline: 3.9986x; 1.0016x over previous
import functools

import jax
import jax.numpy as jnp
from jax.experimental import pallas as pl
from jax.experimental.pallas import tpu as pltpu


def _round_up(x, m):
    return (x + m - 1) // m * m


def _bits(maxval):
    return [1 << j for j in reversed(range(max(1, int(maxval).bit_length())))]


def _start_or_wait(plans, dst_ref, sems, start):
    for idx, (cond, src, s, d, b) in enumerate(plans):
        @pl.when(cond)
        def _(src=src, s=s, d=d, b=b, idx=idx):
            cp = pltpu.make_async_copy(
                src.at[pl.ds(s, b)], dst_ref.at[pl.ds(d, b)], sems.at[idx])
            if start:
                cp.start()
            else:
                cp.wait()


def _pack_body(soff_ref, len_ref, doff_ref, src_ref, out_ref, sems, *,
               nseg, bits):
    plans = []
    for i in range(nseg):
        nrows = len_ref[i]
        cnt = nrows + (nrows & 1)
        s = soff_ref[i]
        d = doff_ref[i]
        for b in bits:
            take = cnt & b
            plans.append(((take != 0), src_ref, s, d, b))
            s = s + take
            d = d + take
    _start_or_wait(plans, out_ref, sems, True)
    _start_or_wait(plans, out_ref, sems, False)


def _level_body(off_ref, len_ref, noff_ref, x_ref, w1_ref, b1_ref, w2_ref,
                b2_ref, g_ref, bb_ref, buf_ref, nbuf_ref, y3, sems, lsems, *,
                nseg, tp, bits, sub, lanes):
    x = x_ref[...].reshape(tp, 2 * sub * lanes)
    h = jax.lax.dot_general(x, w1_ref[...], (((1,), (1,)), ((), ())),
                            preferred_element_type=jnp.float32)
    h = jnp.maximum(h + b1_ref[...], 0.0)
    y = jax.lax.dot_general(h, w2_ref[...], (((1,), (1,)), ((), ())),
                            preferred_element_type=jnp.float32)
    y = y + b2_ref[...]
    mu = jnp.mean(y, axis=-1, keepdims=True)
    var = jnp.mean((y - mu) ** 2, axis=-1, keepdims=True)
    y = (y - mu) * jax.lax.rsqrt(var + 1e-5) * g_ref[...] + bb_ref[...]

    for s in range(sub):
        y3[:, s, :] = y[:, s * lanes:(s + 1) * lanes]

    i = pl.program_id(0)
    t0 = i * tp
    plans = []
    for s in range(nseg):
        po = off_ref[s] // 2
        pairs = len_ref[s] // 2
        odd = len_ref[s] & 1
        a = jnp.maximum(t0, po)
        b = jnp.minimum(t0 + tp, po + pairs)
        cnt = jnp.maximum(b - a, 0)
        src = a - t0
        d = noff_ref[s] + odd + (a - po)
        for bit in bits:
            take = cnt & bit
            plans.append(((take != 0), y3, src, d, bit))
            src = src + take
            d = d + take
    _start_or_wait(plans, nbuf_ref, sems, True)

    lplans = []
    for s in range(nseg):
        ln = len_ref[s]
        lplans.append(((ln & 1) == 1, buf_ref,
                       jnp.maximum(off_ref[s] + ln - 1, 0), noff_ref[s], 1))

    @pl.when(i == 0)
    def _():
        _start_or_wait(lplans, nbuf_ref, lsems, True)
        _start_or_wait(lplans, nbuf_ref, lsems, False)

    _start_or_wait(plans, nbuf_ref, sems, False)


def _run_level(buf3, w1, b1, w2, b2, g, bb, offs, lens, noffs,
               out_rows, tp, sub, lanes):
    rows = buf3.shape[0]
    p = rows // 2
    dim = sub * lanes
    d2 = 2 * dim
    grid = (p + tp - 1) // tp
    bits = _bits(tp)
    nseg = lens.shape[0]
    return pl.pallas_call(
        functools.partial(_level_body, nseg=nseg, tp=tp, bits=bits,
                          sub=sub, lanes=lanes),
        grid=(grid,),
        in_specs=[
            pl.BlockSpec(memory_space=pltpu.SMEM),
            pl.BlockSpec(memory_space=pltpu.SMEM),
            pl.BlockSpec(memory_space=pltpu.SMEM),
            pl.BlockSpec((2 * tp, sub, lanes), lambda i: (i, 0, 0)),
            pl.BlockSpec((d2, d2), lambda i: (0, 0)),
            pl.BlockSpec((1, d2), lambda i: (0, 0)),
            pl.BlockSpec((dim, d2), lambda i: (0, 0)),
            pl.BlockSpec((1, dim), lambda i: (0, 0)),
            pl.BlockSpec((1, dim), lambda i: (0, 0)),
            pl.BlockSpec((1, dim), lambda i: (0, 0)),
            pl.BlockSpec(memory_space=pl.ANY),
        ],
        out_specs=pl.BlockSpec(memory_space=pl.ANY),
        out_shape=jax.ShapeDtypeStruct((out_rows, sub, lanes), jnp.float32),
        scratch_shapes=[pltpu.VMEM((tp, sub, lanes), jnp.float32),
                        pltpu.SemaphoreType.DMA((nseg * len(bits),)),
                        pltpu.SemaphoreType.DMA((nseg,))],
    )(offs, lens, noffs, buf3, w1, b1.reshape(1, d2), w2,
      b2.reshape(1, dim), g.reshape(1, dim), bb.reshape(1, dim), buf3)


def _deep_body(offs_ref, lens_ref, buf5_ref, w1_ref, b1_ref, w2_ref, b2_ref,
               g_ref, bb_ref, arena_ref, x3, y3, csem, sems, lsems, *,
               nseg, slot, pmax, bits, sub, lanes):
    j = pl.program_id(0)

    @pl.when(j == 0)
    def _():
        pltpu.make_async_copy(buf5_ref, arena_ref.at[pl.ds(0, slot)],
                              csem).start()
        pltpu.make_async_copy(buf5_ref, arena_ref.at[pl.ds(0, slot)],
                              csem).wait()

    pltpu.make_async_copy(arena_ref.at[pl.ds(j * slot, slot)], x3,
                          csem).start()
    pltpu.make_async_copy(arena_ref.at[pl.ds(j * slot, slot)], x3,
                          csem).wait()

    x = x3[...].reshape(pmax, 2 * sub * lanes)
    h = jax.lax.dot_general(x, w1_ref[...], (((1,), (1,)), ((), ())),
                            preferred_element_type=jnp.float32)
    h = jnp.maximum(h + b1_ref[...], 0.0)
    y = jax.lax.dot_general(h, w2_ref[...], (((1,), (1,)), ((), ())),
                            preferred_element_type=jnp.float32)
    y = y + b2_ref[...]
    mu = jnp.mean(y, axis=-1, keepdims=True)
    var = jnp.mean((y - mu) ** 2, axis=-1, keepdims=True)
    y = (y - mu) * jax.lax.rsqrt(var + 1e-5) * g_ref[...] + bb_ref[...]
    for s in range(sub):
        y3[:, s, :] = y[:, s * lanes:(s + 1) * lanes]

    dbase = (j + 1) * slot
    plans = []
    for s in range(nseg):
        ln = lens_ref[j, s]
        po = offs_ref[j, s] // 2
        pairs = ln // 2
        odd = ln & 1
        src = po
        d = dbase + offs_ref[j + 1, s] + odd
        for bit in bits:
            take = pairs & bit
            plans.append(((take != 0), y3, src, d, bit))
            src = src + take
            d = d + take
        plans.append(((ln & 1) == 1, arena_ref,
                      j * slot + jnp.maximum(offs_ref[j, s] + ln - 1, 0),
                      dbase + offs_ref[j + 1, s], 1))
    _start_or_wait(plans, arena_ref, sems, True)
    _start_or_wait(plans, arena_ref, sems, False)


def _run_deep(buf5, w1, b1, w2, b2, g, bb, offs_all, lens_all, depth,
              slot, sub, lanes):
    pmax = slot // 2
    dim = sub * lanes
    d2 = 2 * dim
    bits = _bits(pmax)
    nseg = lens_all.shape[1]
    smem = pl.BlockSpec(memory_space=pltpu.SMEM)
    return pl.pallas_call(
        functools.partial(_deep_body, nseg=nseg, slot=slot, pmax=pmax,
                          bits=bits, sub=sub, lanes=lanes),
        grid=(depth,),
        in_specs=[
            smem, smem,
            pl.BlockSpec(memory_space=pl.ANY),
            pl.BlockSpec((d2, d2), lambda j: (0, 0)),
            pl.BlockSpec((1, d2), lambda j: (0, 0)),
            pl.BlockSpec((dim, d2), lambda j: (0, 0)),
            pl.BlockSpec((1, dim), lambda j: (0, 0)),
            pl.BlockSpec((1, dim), lambda j: (0, 0)),
            pl.BlockSpec((1, dim), lambda j: (0, 0)),
        ],
        out_specs=pl.BlockSpec(memory_space=pl.ANY),
        out_shape=jax.ShapeDtypeStruct(((depth + 1) * slot, sub, lanes),
                                       jnp.float32),
        scratch_shapes=[pltpu.VMEM((slot, sub, lanes), jnp.float32),
                        pltpu.VMEM((pmax, sub, lanes), jnp.float32),
                        pltpu.SemaphoreType.DMA,
                        pltpu.SemaphoreType.DMA((nseg * (len(bits) + 1),)),
                        pltpu.SemaphoreType.DMA((nseg,))],
    )(offs_all, lens_all, buf5, w1, b1.reshape(1, d2), w2,
      b2.reshape(1, dim), g.reshape(1, dim), bb.reshape(1, dim))


def _extract_body(off_ref, len_ref, buf_ref, out_ref, scratch, sems, *, nseg):
    for i in range(nseg):
        pltpu.make_async_copy(buf_ref.at[pl.ds(off_ref[i], 1)],
                              scratch.at[pl.ds(i, 1)], sems.at[i]).start()
    for i in range(nseg):
        pltpu.make_async_copy(buf_ref.at[pl.ds(off_ref[i], 1)],
                              scratch.at[pl.ds(i, 1)], sems.at[i]).wait()
    for i in range(nseg):
        nz = len_ref[i] > 0

        @pl.when(nz)
        def _(i=i):
            out_ref[i] = scratch[i]

        @pl.when(jnp.logical_not(nz))
        def _(i=i):
            out_ref[i] = jnp.zeros_like(scratch[i])


def kernel(args, limits, W1, b1, W2, b2, ln_g, ln_b):
    n, dim = args.shape
    d2 = 2 * dim
    nseg = limits.shape[0] - 1
    levels = max(1, (n - 1).bit_length())
    maxlen = n - 1

    limits = limits.astype(jnp.int32)
    lens0 = limits[1:] - limits[:-1]

    lvl_lens, lvl_offs, lvl_sizes = [], [], []
    ln = lens0
    for k in range(levels + 1):
        padded = ln + (ln & 1)
        offs = jnp.concatenate(
            [jnp.zeros((1,), jnp.int32), jnp.cumsum(padded)[:-1].astype(jnp.int32)])
        lvl_lens.append(ln)
        lvl_offs.append(offs)
        lvl_sizes.append(_round_up(-(-maxlen // (1 << k)) + 2 * nseg, 16))
        ln = (ln + 1) // 2

    anyspec = pl.BlockSpec(memory_space=pl.ANY)
    smem = pl.BlockSpec(memory_space=pltpu.SMEM)

    lanes = 128 if dim % 128 == 0 else 1
    sub = dim // lanes

    bits0 = _bits(lvl_sizes[0])
    buf = pl.pallas_call(
        functools.partial(_pack_body, nseg=nseg, bits=bits0),
        in_specs=[smem, smem, smem, anyspec],
        out_specs=anyspec,
        out_shape=jax.ShapeDtypeStruct((lvl_sizes[0], sub, lanes), jnp.float32),
        scratch_shapes=[pltpu.SemaphoreType.DMA((nseg * len(bits0),))],
    )(limits[:-1], lens0, lvl_offs[0], args.reshape(n, sub, lanes))

    kd = levels
    for k in range(levels):
        if lvl_sizes[k] <= 272:
            kd = k
            break
    for k in range(kd):
        p = lvl_sizes[k] // 2
        tp = min(256, p)
        buf = _run_level(buf, W1, b1, W2, b2, ln_g, ln_b,
                         lvl_offs[k], lvl_lens[k], lvl_offs[k + 1],
                         lvl_sizes[k + 1], tp, sub, lanes)

    depth = levels - kd
    if depth > 0:
        slot = lvl_sizes[kd]
        offs_all = jnp.stack(lvl_offs[kd:levels + 1])
        lens_all = jnp.stack(lvl_lens[kd:levels])
        buf = _run_deep(buf, W1, b1, W2, b2, ln_g, ln_b, offs_all, lens_all,
                        depth, slot, sub, lanes)
        final_offs = lvl_offs[levels] + depth * slot
    else:
        final_offs = lvl_offs[levels]

    out = pl.pallas_call(
        functools.partial(_extract_body, nseg=nseg),
        in_specs=[smem, smem, anyspec],
        out_shape=jax.ShapeDtypeStruct((nseg, sub, lanes), jnp.float32),
        scratch_shapes=[pltpu.VMEM((nseg, sub, lanes), jnp.float32),
                        pltpu.SemaphoreType.DMA((nseg,))],
    )(final_offs, lens0, buf)
    return out.reshape(nseg, dim)


# speedup vs baseline: 4.1964x; 1.0495x over previous
import functools

import jax
import jax.numpy as jnp
from jax.experimental import pallas as pl
from jax.experimental.pallas import tpu as pltpu


def _round_up(x, m):
    return (x + m - 1) // m * m


def _bits(maxval):
    return [1 << j for j in reversed(range(max(1, int(maxval).bit_length())))]


def _start_or_wait(plans, dst_ref, sems, start):
    for idx, (cond, src, s, d, b) in enumerate(plans):
        @pl.when(cond)
        def _(src=src, s=s, d=d, b=b, idx=idx):
            cp = pltpu.make_async_copy(
                src.at[pl.ds(s, b)], dst_ref.at[pl.ds(d, b)], sems.at[idx])
            if start:
                cp.start()
            else:
                cp.wait()


def _pack_body(soff_ref, len_ref, doff_ref, src_ref, out_ref, sems, *,
               nseg, bits):
    plans = []
    for i in range(nseg):
        nrows = len_ref[i]
        cnt = nrows + (nrows & 1)
        s = soff_ref[i]
        d = doff_ref[i]
        for b in bits:
            take = cnt & b
            plans.append(((take != 0), src_ref, s, d, b))
            s = s + take
            d = d + take
    _start_or_wait(plans, out_ref, sems, True)
    _start_or_wait(plans, out_ref, sems, False)


def _level_body(off_ref, len_ref, noff_ref, up_ref, x_ref, w1_ref, b1_ref,
                w2_ref, b2_ref, g_ref, bb_ref, buf_ref, nbuf_ref, y3, sems,
                lsems, *, nseg, tp, bits, sub, lanes):
    i = pl.program_id(0)
    t0 = i * tp

    @pl.when(t0 < up_ref[0])
    def _():
        x = x_ref[...].reshape(tp, 2 * sub * lanes)
        h = jax.lax.dot_general(x, w1_ref[...], (((1,), (1,)), ((), ())),
                                preferred_element_type=jnp.float32)
        h = jnp.maximum(h + b1_ref[...], 0.0)
        y = jax.lax.dot_general(h, w2_ref[...], (((1,), (1,)), ((), ())),
                                preferred_element_type=jnp.float32)
        y = y + b2_ref[...]
        mu = jnp.mean(y, axis=-1, keepdims=True)
        var = jnp.mean((y - mu) ** 2, axis=-1, keepdims=True)
        y = (y - mu) * jax.lax.rsqrt(var + 1e-5) * g_ref[...] + bb_ref[...]
        for s in range(sub):
            y3[:, s, :] = y[:, s * lanes:(s + 1) * lanes]
    plans = []
    for s in range(nseg):
        po = off_ref[s] // 2
        pairs = len_ref[s] // 2
        odd = len_ref[s] & 1
        a = jnp.maximum(t0, po)
        b = jnp.minimum(t0 + tp, po + pairs)
        cnt = jnp.maximum(b - a, 0)
        src = a - t0
        d = noff_ref[s] + odd + (a - po)
        for bit in bits:
            take = cnt & bit
            plans.append(((take != 0), y3, src, d, bit))
            src = src + take
            d = d + take
    _start_or_wait(plans, nbuf_ref, sems, True)

    lplans = []
    for s in range(nseg):
        ln = len_ref[s]
        lplans.append(((ln & 1) == 1, buf_ref,
                       jnp.maximum(off_ref[s] + ln - 1, 0), noff_ref[s], 1))

    @pl.when(i == 0)
    def _():
        _start_or_wait(lplans, nbuf_ref, lsems, True)
        _start_or_wait(lplans, nbuf_ref, lsems, False)

    _start_or_wait(plans, nbuf_ref, sems, False)


def _run_level(buf3, w1, b1, w2, b2, g, bb, offs, lens, noffs, upairs,
               out_rows, tp, sub, lanes):
    rows = buf3.shape[0]
    p = rows // 2
    dim = sub * lanes
    d2 = 2 * dim
    grid = (p + tp - 1) // tp
    bits = _bits(tp)
    nseg = lens.shape[0]
    return pl.pallas_call(
        functools.partial(_level_body, nseg=nseg, tp=tp, bits=bits,
                          sub=sub, lanes=lanes),
        grid=(grid,),
        in_specs=[
            pl.BlockSpec(memory_space=pltpu.SMEM),
            pl.BlockSpec(memory_space=pltpu.SMEM),
            pl.BlockSpec(memory_space=pltpu.SMEM),
            pl.BlockSpec(memory_space=pltpu.SMEM),
            pl.BlockSpec((2 * tp, sub, lanes), lambda i: (i, 0, 0)),
            pl.BlockSpec((d2, d2), lambda i: (0, 0)),
            pl.BlockSpec((1, d2), lambda i: (0, 0)),
            pl.BlockSpec((dim, d2), lambda i: (0, 0)),
            pl.BlockSpec((1, dim), lambda i: (0, 0)),
            pl.BlockSpec((1, dim), lambda i: (0, 0)),
            pl.BlockSpec((1, dim), lambda i: (0, 0)),
            pl.BlockSpec(memory_space=pl.ANY),
        ],
        out_specs=pl.BlockSpec(memory_space=pl.ANY),
        out_shape=jax.ShapeDtypeStruct((out_rows, sub, lanes), jnp.float32),
        scratch_shapes=[pltpu.VMEM((tp, sub, lanes), jnp.float32),
                        pltpu.SemaphoreType.DMA((nseg * len(bits),)),
                        pltpu.SemaphoreType.DMA((nseg,))],
    )(offs, lens, noffs, upairs, buf3, w1, b1.reshape(1, d2), w2,
      b2.reshape(1, dim), g.reshape(1, dim), bb.reshape(1, dim), buf3)


def _deep_body(offs_ref, lens_ref, buf5_ref, w1_ref, b1_ref, w2_ref, b2_ref,
               g_ref, bb_ref, arena_ref, x3, y3, csem, sems, lsems, *,
               nseg, slot, pmax, bits, sub, lanes):
    j = pl.program_id(0)

    @pl.when(j == 0)
    def _():
        pltpu.make_async_copy(buf5_ref, arena_ref.at[pl.ds(0, slot)],
                              csem).start()
        pltpu.make_async_copy(buf5_ref, arena_ref.at[pl.ds(0, slot)],
                              csem).wait()

    pltpu.make_async_copy(arena_ref.at[pl.ds(j * slot, slot)], x3,
                          csem).start()
    pltpu.make_async_copy(arena_ref.at[pl.ds(j * slot, slot)], x3,
                          csem).wait()

    x = x3[...].reshape(pmax, 2 * sub * lanes)
    h = jax.lax.dot_general(x, w1_ref[...], (((1,), (1,)), ((), ())),
                            preferred_element_type=jnp.float32)
    h = jnp.maximum(h + b1_ref[...], 0.0)
    y = jax.lax.dot_general(h, w2_ref[...], (((1,), (1,)), ((), ())),
                            preferred_element_type=jnp.float32)
    y = y + b2_ref[...]
    mu = jnp.mean(y, axis=-1, keepdims=True)
    var = jnp.mean((y - mu) ** 2, axis=-1, keepdims=True)
    y = (y - mu) * jax.lax.rsqrt(var + 1e-5) * g_ref[...] + bb_ref[...]
    for s in range(sub):
        y3[:, s, :] = y[:, s * lanes:(s + 1) * lanes]

    dbase = (j + 1) * slot
    plans = []
    for s in range(nseg):
        ln = lens_ref[j, s]
        po = offs_ref[j, s] // 2
        pairs = ln // 2
        odd = ln & 1
        src = po
        d = dbase + offs_ref[j + 1, s] + odd
        for bit in bits:
            take = pairs & bit
            plans.append(((take != 0), y3, src, d, bit))
            src = src + take
            d = d + take
        plans.append(((ln & 1) == 1, arena_ref,
                      j * slot + jnp.maximum(offs_ref[j, s] + ln - 1, 0),
                      dbase + offs_ref[j + 1, s], 1))
    _start_or_wait(plans, arena_ref, sems, True)
    _start_or_wait(plans, arena_ref, sems, False)


def _run_deep(buf5, w1, b1, w2, b2, g, bb, offs_all, lens_all, depth,
              slot, sub, lanes):
    pmax = slot // 2
    dim = sub * lanes
    d2 = 2 * dim
    bits = _bits(pmax)
    nseg = lens_all.shape[1]
    smem = pl.BlockSpec(memory_space=pltpu.SMEM)
    return pl.pallas_call(
        functools.partial(_deep_body, nseg=nseg, slot=slot, pmax=pmax,
                          bits=bits, sub=sub, lanes=lanes),
        grid=(depth,),
        in_specs=[
            smem, smem,
            pl.BlockSpec(memory_space=pl.ANY),
            pl.BlockSpec((d2, d2), lambda j: (0, 0)),
            pl.BlockSpec((1, d2), lambda j: (0, 0)),
            pl.BlockSpec((dim, d2), lambda j: (0, 0)),
            pl.BlockSpec((1, dim), lambda j: (0, 0)),
            pl.BlockSpec((1, dim), lambda j: (0, 0)),
            pl.BlockSpec((1, dim), lambda j: (0, 0)),
        ],
        out_specs=pl.BlockSpec(memory_space=pl.ANY),
        out_shape=jax.ShapeDtypeStruct(((depth + 1) * slot, sub, lanes),
                                       jnp.float32),
        scratch_shapes=[pltpu.VMEM((slot, sub, lanes), jnp.float32),
                        pltpu.VMEM((pmax, sub, lanes), jnp.float32),
                        pltpu.SemaphoreType.DMA,
                        pltpu.SemaphoreType.DMA((nseg * (len(bits) + 1),)),
                        pltpu.SemaphoreType.DMA((nseg,))],
    )(offs_all, lens_all, buf5, w1, b1.reshape(1, d2), w2,
      b2.reshape(1, dim), g.reshape(1, dim), bb.reshape(1, dim))


def _extract_body(off_ref, len_ref, buf_ref, out_ref, scratch, sems, *, nseg):
    for i in range(nseg):
        pltpu.make_async_copy(buf_ref.at[pl.ds(off_ref[i], 1)],
                              scratch.at[pl.ds(i, 1)], sems.at[i]).start()
    for i in range(nseg):
        pltpu.make_async_copy(buf_ref.at[pl.ds(off_ref[i], 1)],
                              scratch.at[pl.ds(i, 1)], sems.at[i]).wait()
    for i in range(nseg):
        nz = len_ref[i] > 0

        @pl.when(nz)
        def _(i=i):
            out_ref[i] = scratch[i]

        @pl.when(jnp.logical_not(nz))
        def _(i=i):
            out_ref[i] = jnp.zeros_like(scratch[i])


def kernel(args, limits, W1, b1, W2, b2, ln_g, ln_b):
    n, dim = args.shape
    d2 = 2 * dim
    nseg = limits.shape[0] - 1
    levels = max(1, (n - 1).bit_length())
    maxlen = n - 1

    limits = limits.astype(jnp.int32)
    lens0 = limits[1:] - limits[:-1]

    lvl_lens, lvl_offs, lvl_sizes, lvl_upairs = [], [], [], []
    ln = lens0
    for k in range(levels + 1):
        padded = ln + (ln & 1)
        csum = jnp.cumsum(padded).astype(jnp.int32)
        offs = jnp.concatenate([jnp.zeros((1,), jnp.int32), csum[:-1]])
        lvl_lens.append(ln)
        lvl_offs.append(offs)
        lvl_sizes.append(_round_up(-(-maxlen // (1 << k)) + 2 * nseg, 16))
        lvl_upairs.append((csum[-1:] + 1) // 2)
        ln = (ln + 1) // 2

    anyspec = pl.BlockSpec(memory_space=pl.ANY)
    smem = pl.BlockSpec(memory_space=pltpu.SMEM)

    lanes = 128 if dim % 128 == 0 else 1
    sub = dim // lanes

    bits0 = _bits(lvl_sizes[0])
    buf = pl.pallas_call(
        functools.partial(_pack_body, nseg=nseg, bits=bits0),
        in_specs=[smem, smem, smem, anyspec],
        out_specs=anyspec,
        out_shape=jax.ShapeDtypeStruct((lvl_sizes[0], sub, lanes), jnp.float32),
        scratch_shapes=[pltpu.SemaphoreType.DMA((nseg * len(bits0),))],
    )(limits[:-1], lens0, lvl_offs[0], args.reshape(n, sub, lanes))

    kd = levels
    for k in range(levels):
        if lvl_sizes[k] <= 272:
            kd = k
            break
    for k in range(kd):
        p = lvl_sizes[k] // 2
        tp = min(512, p)
        buf = _run_level(buf, W1, b1, W2, b2, ln_g, ln_b,
                         lvl_offs[k], lvl_lens[k], lvl_offs[k + 1],
                         lvl_upairs[k], lvl_sizes[k + 1], tp, sub, lanes)

    depth = levels - kd
    if depth > 0:
        slot = lvl_sizes[kd]
        offs_all = jnp.stack(lvl_offs[kd:levels + 1])
        lens_all = jnp.stack(lvl_lens[kd:levels])
        buf = _run_deep(buf, W1, b1, W2, b2, ln_g, ln_b, offs_all, lens_all,
                        depth, slot, sub, lanes)
        final_offs = lvl_offs[levels] + depth * slot
    else:
        final_offs = lvl_offs[levels]

    out = pl.pallas_call(
        functools.partial(_extract_body, nseg=nseg),
        in_specs=[smem, smem, anyspec],
        out_shape=jax.ShapeDtypeStruct((nseg, sub, lanes), jnp.float32),
        scratch_shapes=[pltpu.VMEM((nseg, sub, lanes), jnp.float32),
                        pltpu.SemaphoreType.DMA((nseg,))],
    )(final_offs, lens0, buf)
    return out.reshape(nseg, dim)
